# Initial kernel scaffold; baseline (speedup 1.0000x reference)
import jax
import jax.numpy as jnp
from jax.experimental import pallas as pl


def kernel(h, edge_index, W, b, gamma, beta):
    raise NotImplementedError("write your pallas kernel here")


# speedup vs baseline: 8.2669x; 8.2669x over previous
import functools

import jax
import jax.numpy as jnp
from jax import lax
from jax.experimental import pallas as pl
from jax.experimental.pallas import tpu as pltpu
from jax.experimental.pallas import tpu_sc as plsc

N = 10000
E = 320000
DIN = 128
DOUT = 128

NC = 2
NS = 16
NT = NC * NS
EPT = E // NT
CHUNK = 80
NCHUNK = EPT // CHUNK
ACC_ROWS = 10240
ROWS_PT = ACC_ROWS // NS
ZROWS = 32
DEG_PAD = 10240
DEGS_PT = DEG_PAD // NS


def _sc_aggregate(h, u3, v3):
    mesh = plsc.VectorSubcoreMesh(core_axis_name="c", subcore_axis_name="s")

    @functools.partial(
        pl.kernel,
        out_type=[
            jax.ShapeDtypeStruct((NC, ACC_ROWS, DIN), jnp.float32),
            jax.ShapeDtypeStruct((NC, DEG_PAD), jnp.float32),
        ],
        mesh=mesh,
        scratch_types=[
            pltpu.VMEM_SHARED((ACC_ROWS, DIN), jnp.float32),
            pltpu.VMEM_SHARED((DEG_PAD,), jnp.float32),
            pltpu.VMEM((NCHUNK, CHUNK), jnp.int32),
            pltpu.VMEM((NCHUNK, CHUNK), jnp.int32),
            pltpu.VMEM((CHUNK, DIN), jnp.float32),
            pltpu.VMEM((CHUNK,), jnp.float32),
            pltpu.VMEM((ZROWS, DIN), jnp.float32),
            pltpu.VMEM((DEGS_PT,), jnp.float32),
        ],
    )
    def agg(h_hbm, u_hbm, v_hbm, sum_out, deg_out,
            acc, dacc, ub, vb, rows, ones, zrows, zdeg):
        c = lax.axis_index("c")
        s = lax.axis_index("s")
        wid = c * NS + s

        z16 = jnp.zeros((16,), jnp.float32)
        o16 = jnp.ones((16,), jnp.float32)

        @pl.loop(0, ZROWS)
        def _(i):
            for j in range(DIN // 16):
                zrows[i, pl.ds(j * 16, 16)] = z16

        @pl.loop(0, DEGS_PT // 16)
        def _(i):
            zdeg[pl.ds(i * 16, 16)] = z16

        for j in range(CHUNK // 16):
            ones[pl.ds(j * 16, 16)] = o16

        for t in range(ROWS_PT // ZROWS):
            pltpu.sync_copy(zrows, acc.at[pl.ds(s * ROWS_PT + t * ZROWS, ZROWS)])
        pltpu.sync_copy(zdeg, dacc.at[pl.ds(s * DEGS_PT, DEGS_PT)])

        pltpu.sync_copy(u_hbm.at[wid], ub)
        pltpu.sync_copy(v_hbm.at[wid], vb)

        plsc.subcore_barrier()

        @pl.loop(0, NCHUNK)
        def _(j):
            pltpu.sync_copy(h_hbm.at[ub.at[j]], rows)
            pltpu.sync_copy(rows, acc.at[vb.at[j]], add=True)
            pltpu.sync_copy(ones, dacc.at[vb.at[j]], add=True)

        plsc.subcore_barrier()

        for t in range(ROWS_PT // ZROWS):
            r0 = s * ROWS_PT + t * ZROWS
            pltpu.sync_copy(acc.at[pl.ds(r0, ZROWS)], zrows)
            pltpu.sync_copy(zrows, sum_out.at[c, pl.ds(r0, ZROWS)])
        pltpu.sync_copy(dacc.at[pl.ds(s * DEGS_PT, DEGS_PT)], zdeg)
        pltpu.sync_copy(zdeg, deg_out.at[c, pl.ds(s * DEGS_PT, DEGS_PT)])

    return agg(h, u3, v3)


def _tc_update(h, part_sums, part_degs, W, b, gamma, beta):
    BLK = 400

    def body(h_ref, s_ref, d_ref, w_ref, b_ref, g_ref, be_ref, o_ref):
        hb = h_ref[...]
        sm = s_ref[0] + s_ref[1]
        dg = jnp.maximum(d_ref[0] + d_ref[1], 1.0)
        mean = sm / dg
        out = jnp.dot(hb, w_ref[:DIN, :], preferred_element_type=jnp.float32)
        out = out + jnp.dot(mean, w_ref[DIN:, :], preferred_element_type=jnp.float32)
        out = out + b_ref[...]
        mu = jnp.mean(out, axis=-1, keepdims=True)
        var = jnp.mean((out - mu) ** 2, axis=-1, keepdims=True)
        y = (out - mu) * lax.rsqrt(var + 1e-5)
        y = y * g_ref[...] + be_ref[...]
        o_ref[...] = 0.5 * y * (1.0 + lax.erf(y * 0.7071067811865476))

    return pl.pallas_call(
        body,
        grid=(N // BLK,),
        in_specs=[
            pl.BlockSpec((BLK, DIN), lambda i: (i, 0)),
            pl.BlockSpec((NC, BLK, DIN), lambda i: (0, i, 0)),
            pl.BlockSpec((NC, BLK, 1), lambda i: (0, i, 0)),
            pl.BlockSpec((2 * DIN, DOUT), lambda i: (0, 0)),
            pl.BlockSpec((1, DOUT), lambda i: (0, 0)),
            pl.BlockSpec((1, DOUT), lambda i: (0, 0)),
            pl.BlockSpec((1, DOUT), lambda i: (0, 0)),
        ],
        out_specs=pl.BlockSpec((BLK, DOUT), lambda i: (i, 0)),
        out_shape=jax.ShapeDtypeStruct((N, DOUT), jnp.float32),
    )(h, part_sums, part_degs, W, b, gamma, beta)


def kernel(h, edge_index, W, b, gamma, beta):
    u3 = edge_index[0].reshape(NT, NCHUNK, CHUNK)
    v3 = edge_index[1].reshape(NT, NCHUNK, CHUNK)
    part_sums, degp = _sc_aggregate(h, u3, v3)
    part_degs = degp.reshape(NC, DEG_PAD, 1)
    return _tc_update(
        h, part_sums, part_degs,
        W, b.reshape(1, DOUT), gamma.reshape(1, DOUT), beta.reshape(1, DOUT),
    )
